# Initial kernel scaffold; baseline (speedup 1.0000x reference)
import jax
import jax.numpy as jnp
from jax.experimental import pallas as pl


def kernel(ori_feats, W, attn_l, attn_r, bias):
    raise NotImplementedError("write your pallas kernel here")


# speedup vs baseline: 31.3007x; 31.3007x over previous
import jax
import jax.numpy as jnp
from jax.experimental import pallas as pl
from jax.experimental.pallas import tpu as pltpu

N_FEATURES = 128
WINDOW = 10
T = 4096
PAD_ROWS = T + 16


def _gat_body(padded_ref, w_ref, al_ref, ar_ref, bias_ref, out_ref):
    h = jnp.dot(padded_ref[...], w_ref[...], preferred_element_type=jnp.float32)
    el = jnp.sum(h * al_ref[...], axis=1, keepdims=True)
    er = jnp.sum(h * ar_ref[...], axis=1, keepdims=True)
    er9 = jax.lax.slice(er, (WINDOW - 1, 0), (WINDOW - 1 + T, 1))

    scores = []
    for i in range(WINDOW):
        eli = jax.lax.slice(el, (i, 0), (i + T, 1))
        s = eli + er9
        scores.append(jnp.where(s > 0, s, 0.2 * s))
    m = scores[0]
    for i in range(1, WINDOW):
        m = jnp.maximum(m, scores[i])
    ees = [jnp.exp(s - m) for s in scores]
    denom = ees[0]
    for i in range(1, WINDOW):
        denom = denom + ees[i]
    inv = 1.0 / denom

    acc = jnp.broadcast_to(bias_ref[...], (T, N_FEATURES))
    for i in range(WINDOW):
        hi = jax.lax.slice(h, (i, 0), (i + T, N_FEATURES))
        acc = acc + (ees[i] * inv) * hi
    out_ref[...] = acc


def kernel(ori_feats, W, attn_l, attn_r, bias):
    pad = jnp.broadcast_to(ori_feats[0:1], (WINDOW - 1, N_FEATURES))
    tail = jnp.zeros((PAD_ROWS - T - (WINDOW - 1), N_FEATURES), jnp.float32)
    padded = jnp.concatenate([pad, ori_feats, tail], axis=0)

    out = pl.pallas_call(
        _gat_body,
        out_shape=jax.ShapeDtypeStruct((T, N_FEATURES), jnp.float32),
        in_specs=[
            pl.BlockSpec(memory_space=pltpu.VMEM),
            pl.BlockSpec(memory_space=pltpu.VMEM),
            pl.BlockSpec(memory_space=pltpu.VMEM),
            pl.BlockSpec(memory_space=pltpu.VMEM),
            pl.BlockSpec(memory_space=pltpu.VMEM),
        ],
        out_specs=pl.BlockSpec(memory_space=pltpu.VMEM),
    )(padded, W, attn_l.reshape(1, N_FEATURES), attn_r.reshape(1, N_FEATURES),
      bias.reshape(1, N_FEATURES))
    return out[:, None, :]
